# baseline (speedup 1.0000x reference)
import numpy as np
import jax
import jax.numpy as jnp
from jax import lax
from jax.experimental import pallas as pl
from jax.experimental.pallas import tpu as pltpu
from jax.experimental.pallas import tpu_sc as plsc

N_NODES = 1024
N_FEAT = 496
INTER = 64
N_EDGES = 65536
ROI = 32

_iu0, _iu1 = np.triu_indices(ROI, k=1)
_S_np = np.zeros((N_FEAT, ROI * ROI), np.float32)
_S_np[np.arange(N_FEAT), _iu0 * ROI + _iu1] = 1.0
_S_np[np.arange(N_FEAT), _iu1 * ROI + _iu0] = 1.0


ROWS_PER_W = N_NODES // 16
E_HALF = N_EDGES // 2
N_CHUNKS = 4
CH = E_HALF // N_CHUNKS
UNROLL = 4


def _sc_count_body(src_hbm, dst_hbm, out_hbm, deg_hbm, src_v, dst_v, acc_v,
                   deg_v):
    cid = lax.axis_index("c")
    sid = lax.axis_index("s")
    lo = sid * ROWS_PER_W
    ebase = cid * E_HALF
    zeros16 = jnp.zeros((16,), jnp.float32)
    ones16 = jnp.ones((16,), jnp.float32)

    def zero_row(i, carry):
        for c in range(N_NODES // 16):
            acc_v[pl.ds(i * N_NODES + c * 16, 16)] = zeros16
        return carry

    lax.fori_loop(0, ROWS_PER_W, zero_row, 0)
    for c in range(ROWS_PER_W // 16):
        deg_v[pl.ds(c * 16, 16)] = zeros16

    for ck in range(N_CHUNKS):
        pltpu.sync_copy(src_hbm.at[pl.ds(ebase + ck * CH, CH)], src_v)
        pltpu.sync_copy(dst_hbm.at[pl.ds(ebase + ck * CH, CH)], dst_v)

        def body(i, carry):
            for u in range(UNROLL):
                s = src_v[pl.ds(i * (16 * UNROLL) + u * 16, 16)]
                d = dst_v[pl.ds(i * (16 * UNROLL) + u * 16, 16)]
                rel = d - lo
                m = (rel >= 0) & (rel < ROWS_PER_W)
                rc = jnp.where(m, rel, 0)
                flat = rc * N_NODES + jnp.where(m, s, 0)
                plsc.addupdate_scatter(acc_v, [flat], ones16, mask=m)
                plsc.addupdate_scatter(deg_v, [rc], ones16, mask=m)
            return carry

        lax.fori_loop(0, CH // (16 * UNROLL), body, 0)

    pltpu.sync_copy(
        acc_v,
        out_hbm.at[pl.ds((cid * N_NODES + lo) * N_NODES,
                         ROWS_PER_W * N_NODES)])
    pltpu.sync_copy(
        deg_v, deg_hbm.at[pl.ds(cid * N_NODES + lo, ROWS_PER_W)])


_SC_COUNT_CACHE = []


def _sc_count(src, dst):
    if not _SC_COUNT_CACHE:
        _SC_COUNT_CACHE.append(pl.kernel(
            _sc_count_body,
            out_type=(
                jax.ShapeDtypeStruct((2 * N_NODES * N_NODES,), jnp.float32),
                jax.ShapeDtypeStruct((2 * N_NODES,), jnp.float32),
            ),
            mesh=plsc.VectorSubcoreMesh(core_axis_name="c", subcore_axis_name="s"),
            compiler_params=pltpu.CompilerParams(needs_layout_passes=False),
            scratch_types=[
                pltpu.VMEM((CH,), jnp.int32),
                pltpu.VMEM((CH,), jnp.int32),
                pltpu.VMEM((ROWS_PER_W * N_NODES,), jnp.float32),
                pltpu.VMEM((ROWS_PER_W,), jnp.float32),
            ],
        ))
    return _SC_COUNT_CACHE[0](src, dst)


RB = 128
NRB = N_NODES // RB


def _dense_body(dcf_ref, xf_ref, w1_ref, b1_ref, x_ref, c0_ref, c1_ref,
                dc_ref, s_ref, z_ref, xs_ref, zs_ref, xws_ref):
    k = pl.program_id(0)

    @pl.when(k == 0)
    def _():
        dinv_all = lax.rsqrt(dcf_ref[...] + 1.0)
        xw = jnp.dot(xf_ref[...], w1_ref[...],
                     preferred_element_type=jnp.float32)
        xws_ref[...] = xw * dinv_all

    C_blk = c0_ref[...] + c1_ref[...]
    dinv_col = lax.rsqrt(dc_ref[...] + 1.0)
    y = (jnp.dot(C_blk, xws_ref[...], preferred_element_type=jnp.float32)
         + xws_ref[pl.ds(k * RB, RB), :]) * dinv_col + b1_ref[...]
    z = jax.nn.sigmoid(y)
    z_ref[...] = z
    S = s_ref[...]
    xs_ref[...] = jnp.dot(x_ref[...], S, preferred_element_type=jnp.float32)
    zs_ref[...] = jnp.dot(z, S, preferred_element_type=jnp.float32)


def _dense(dc, x, W1, b1r, C2, S):
    return pl.pallas_call(
        _dense_body,
        grid=(NRB,),
        in_specs=[
            pl.BlockSpec((N_NODES, 1), lambda k: (0, 0)),
            pl.BlockSpec((N_NODES, N_FEAT), lambda k: (0, 0)),
            pl.BlockSpec((N_FEAT, N_FEAT), lambda k: (0, 0)),
            pl.BlockSpec((1, N_FEAT), lambda k: (0, 0)),
            pl.BlockSpec((RB, N_FEAT), lambda k: (k, 0)),
            pl.BlockSpec((RB, N_NODES), lambda k: (k, 0)),
            pl.BlockSpec((RB, N_NODES), lambda k: (k + NRB, 0)),
            pl.BlockSpec((RB, 1), lambda k: (k, 0)),
            pl.BlockSpec((N_FEAT, ROI * ROI), lambda k: (0, 0)),
        ],
        out_specs=(
            pl.BlockSpec((RB, N_FEAT), lambda k: (k, 0)),
            pl.BlockSpec((RB, ROI * ROI), lambda k: (k, 0)),
            pl.BlockSpec((RB, ROI * ROI), lambda k: (k, 0)),
        ),
        out_shape=(
            jax.ShapeDtypeStruct((N_NODES, N_FEAT), jnp.float32),
            jax.ShapeDtypeStruct((N_NODES, ROI * ROI), jnp.float32),
            jax.ShapeDtypeStruct((N_NODES, ROI * ROI), jnp.float32),
        ),
        scratch_shapes=[pltpu.VMEM((N_NODES, N_FEAT), jnp.float32)],
    )(dc, x, W1, b1r, x, C2, C2, dc, S)


KF = N_NODES * N_FEAT
BK = 31744
NBK = KF // BK


def _gemv_body(zf_ref, wt_ref, bl1_ref, wl2_ref, bl2_ref, s_ref,
               i1_ref, cbt_ref):
    k = pl.program_id(0)

    @pl.when(k == 0)
    def _():
        i1_ref[...] = jnp.zeros_like(i1_ref)

    i1_ref[...] += jax.lax.dot_general(
        wt_ref[...], zf_ref[0],
        dimension_numbers=(((1,), (1,)), ((), ())),
        preferred_element_type=jnp.float32)

    @pl.when(k == NBK - 1)
    def _():
        i1 = jax.nn.sigmoid(i1_ref[...] + bl1_ref[...])
        i1_ref[...] = i1
        i2 = jax.nn.sigmoid(
            jax.lax.dot_general(
                i1, wl2_ref[...],
                dimension_numbers=(((0,), (0,)), ((), ())),
                preferred_element_type=jnp.float32)
            + bl2_ref[...])
        cbt_ref[...] = jnp.dot(i2, s_ref[...],
                               preferred_element_type=jnp.float32)


def _gemv(zf3, Wt, bl1c, Wl2, bl2r, S):
    return pl.pallas_call(
        _gemv_body,
        grid=(NBK,),
        in_specs=[
            pl.BlockSpec((1, 1, BK), lambda k: (k, 0, 0)),
            pl.BlockSpec((INTER, BK), lambda k: (0, k)),
            pl.BlockSpec((INTER, 1), lambda k: (0, 0)),
            pl.BlockSpec((INTER, N_FEAT), lambda k: (0, 0)),
            pl.BlockSpec((1, N_FEAT), lambda k: (0, 0)),
            pl.BlockSpec((N_FEAT, ROI * ROI), lambda k: (0, 0)),
        ],
        out_specs=(
            pl.BlockSpec((INTER, 1), lambda k: (0, 0)),
            pl.BlockSpec((1, ROI * ROI), lambda k: (0, 0)),
        ),
        out_shape=(
            jax.ShapeDtypeStruct((INTER, 1), jnp.float32),
            jax.ShapeDtypeStruct((1, ROI * ROI), jnp.float32),
        ),
    )(zf3, Wt, bl1c, Wl2, bl2r, S)


def kernel(x, edge_index, W1, b1, Wl1, bl1, Wl2, bl2):
    S = jnp.asarray(_S_np)
    src = edge_index[0]
    dst = edge_index[1]
    Cflat, deg2 = _sc_count(src, dst)
    C2 = Cflat.reshape(2 * N_NODES, N_NODES)
    dc = (deg2[:N_NODES] + deg2[N_NODES:]).reshape(N_NODES, 1)
    z, xs, zs = _dense(dc, x, W1, b1.reshape(1, -1), C2, S)
    zf3 = z.reshape(NBK, 1, BK)
    Wt = Wl1.T
    i1, cbt = _gemv(zf3, Wt, bl1.reshape(-1, 1), Wl2,
                    bl2.reshape(1, -1), S)
    x_matrix = xs.reshape(N_NODES, ROI, ROI)
    z_matrix = zs.reshape(N_NODES, ROI, ROI)
    return (x_matrix, z_matrix, z_matrix, i1.reshape(INTER),
            cbt.reshape(ROI, ROI))


# speedup vs baseline: 1.0171x; 1.0171x over previous
import numpy as np
import jax
import jax.numpy as jnp
from jax import lax
from jax.experimental import pallas as pl
from jax.experimental.pallas import tpu as pltpu
from jax.experimental.pallas import tpu_sc as plsc

N_NODES = 1024
N_FEAT = 496
INTER = 64
N_EDGES = 65536
ROI = 32

_iu0, _iu1 = np.triu_indices(ROI, k=1)
_S_np = np.zeros((N_FEAT, ROI * ROI), np.float32)
_S_np[np.arange(N_FEAT), _iu0 * ROI + _iu1] = 1.0
_S_np[np.arange(N_FEAT), _iu1 * ROI + _iu0] = 1.0


ROWS_PER_W = N_NODES // 16
E_HALF = N_EDGES // 2
N_CHUNKS = 4
CH = E_HALF // N_CHUNKS
UNROLL = 4


def _sc_count_body(src_hbm, dst_hbm, out_hbm, deg_hbm, src_v, dst_v, acc_v,
                   deg_v):
    cid = lax.axis_index("c")
    sid = lax.axis_index("s")
    lo = sid * ROWS_PER_W
    ebase = cid * E_HALF
    zeros16 = jnp.zeros((16,), jnp.float32)
    ones16 = jnp.ones((16,), jnp.float32)

    def zero_row(i, carry):
        for c in range(N_NODES // 16):
            acc_v[pl.ds(i * N_NODES + c * 16, 16)] = zeros16
        return carry

    lax.fori_loop(0, ROWS_PER_W, zero_row, 0)
    for c in range(ROWS_PER_W // 16):
        deg_v[pl.ds(c * 16, 16)] = zeros16

    for ck in range(N_CHUNKS):
        pltpu.sync_copy(src_hbm.at[pl.ds(ebase + ck * CH, CH)], src_v)
        pltpu.sync_copy(dst_hbm.at[pl.ds(ebase + ck * CH, CH)], dst_v)

        def body(i, carry):
            for u in range(UNROLL):
                s = src_v[pl.ds(i * (16 * UNROLL) + u * 16, 16)]
                d = dst_v[pl.ds(i * (16 * UNROLL) + u * 16, 16)]
                rel = d - lo
                m = (rel >= 0) & (rel < ROWS_PER_W)
                rc = jnp.where(m, rel, 0)
                flat = rc * N_NODES + jnp.where(m, s, 0)
                plsc.addupdate_scatter(acc_v, [flat], ones16, mask=m)
                plsc.addupdate_scatter(deg_v, [rc], ones16, mask=m)
            return carry

        lax.fori_loop(0, CH // (16 * UNROLL), body, 0)

    pltpu.sync_copy(
        acc_v,
        out_hbm.at[pl.ds((cid * N_NODES + lo) * N_NODES,
                         ROWS_PER_W * N_NODES)])
    pltpu.sync_copy(
        deg_v, deg_hbm.at[pl.ds(cid * N_NODES + lo, ROWS_PER_W)])


_SC_COUNT_CACHE = []


def _sc_count(src, dst):
    if not _SC_COUNT_CACHE:
        _SC_COUNT_CACHE.append(pl.kernel(
            _sc_count_body,
            out_type=(
                jax.ShapeDtypeStruct((2 * N_NODES * N_NODES,), jnp.float32),
                jax.ShapeDtypeStruct((2 * N_NODES,), jnp.float32),
            ),
            mesh=plsc.VectorSubcoreMesh(core_axis_name="c", subcore_axis_name="s"),
            compiler_params=pltpu.CompilerParams(needs_layout_passes=False),
            scratch_types=[
                pltpu.VMEM((CH,), jnp.int32),
                pltpu.VMEM((CH,), jnp.int32),
                pltpu.VMEM((ROWS_PER_W * N_NODES,), jnp.float32),
                pltpu.VMEM((ROWS_PER_W,), jnp.float32),
            ],
        ))
    return _SC_COUNT_CACHE[0](src, dst)


RB = 128
NRB = N_NODES // RB


def _dense_body(dcf_ref, xf_ref, w1_ref, b1_ref, x_ref, c0_ref, c1_ref,
                dc_ref, s_ref, z_ref, xs_ref, zs_ref, xws_ref):
    k = pl.program_id(0)

    @pl.when(k == 0)
    def _():
        dinv_all = lax.rsqrt(dcf_ref[...] + 1.0)
        xw = jnp.dot(xf_ref[...], w1_ref[...],
                     preferred_element_type=jnp.float32)
        xws_ref[...] = xw * dinv_all

    C_blk = c0_ref[...] + c1_ref[...]
    dinv_col = lax.rsqrt(dc_ref[...] + 1.0)
    y = (jnp.dot(C_blk, xws_ref[...], preferred_element_type=jnp.float32)
         + xws_ref[pl.ds(k * RB, RB), :]) * dinv_col + b1_ref[...]
    z = jax.nn.sigmoid(y)
    z_ref[...] = z
    S = s_ref[...]
    xs_ref[...] = jnp.dot(x_ref[...], S, preferred_element_type=jnp.float32)
    zs_ref[...] = jnp.dot(z, S, preferred_element_type=jnp.float32)


def _dense(dc, x, W1, b1r, C2, S):
    return pl.pallas_call(
        _dense_body,
        grid=(NRB,),
        in_specs=[
            pl.BlockSpec((N_NODES, 1), lambda k: (0, 0)),
            pl.BlockSpec((N_NODES, N_FEAT), lambda k: (0, 0)),
            pl.BlockSpec((N_FEAT, N_FEAT), lambda k: (0, 0)),
            pl.BlockSpec((1, N_FEAT), lambda k: (0, 0)),
            pl.BlockSpec((RB, N_FEAT), lambda k: (k, 0)),
            pl.BlockSpec((RB, N_NODES), lambda k: (k, 0)),
            pl.BlockSpec((RB, N_NODES), lambda k: (k + NRB, 0)),
            pl.BlockSpec((RB, 1), lambda k: (k, 0)),
            pl.BlockSpec((N_FEAT, ROI * ROI), lambda k: (0, 0)),
        ],
        out_specs=(
            pl.BlockSpec((RB, N_FEAT), lambda k: (k, 0)),
            pl.BlockSpec((RB, ROI * ROI), lambda k: (k, 0)),
            pl.BlockSpec((RB, ROI * ROI), lambda k: (k, 0)),
        ),
        out_shape=(
            jax.ShapeDtypeStruct((N_NODES, N_FEAT), jnp.float32),
            jax.ShapeDtypeStruct((N_NODES, ROI * ROI), jnp.float32),
            jax.ShapeDtypeStruct((N_NODES, ROI * ROI), jnp.float32),
        ),
        scratch_shapes=[pltpu.VMEM((N_NODES, N_FEAT), jnp.float32)],
    )(dc, x, W1, b1r, x, C2, C2, dc, S)


KF = N_NODES * N_FEAT
NBK = 16
NSTREAM = 4
BK = KF // (NBK * NSTREAM)


def _gemv_body(z0_ref, z1_ref, z2_ref, z3_ref,
               w0_ref, w1_ref, w2_ref, w3_ref,
               bl1_ref, wl2_ref, bl2_ref, s_ref,
               i1_ref, cbt_ref):
    k = pl.program_id(0)

    @pl.when(k == 0)
    def _():
        i1_ref[...] = jnp.zeros_like(i1_ref)

    acc = i1_ref[...]
    for zr, wr in ((z0_ref, w0_ref), (z1_ref, w1_ref),
                   (z2_ref, w2_ref), (z3_ref, w3_ref)):
        acc = acc + jax.lax.dot_general(
            wr[...], zr[0],
            dimension_numbers=(((1,), (1,)), ((), ())),
            preferred_element_type=jnp.float32)
    i1_ref[...] = acc

    @pl.when(k == NBK - 1)
    def _():
        i1 = jax.nn.sigmoid(i1_ref[...] + bl1_ref[...])
        i1_ref[...] = i1
        i2 = jax.nn.sigmoid(
            jax.lax.dot_general(
                i1, wl2_ref[...],
                dimension_numbers=(((0,), (0,)), ((), ())),
                preferred_element_type=jnp.float32)
            + bl2_ref[...])
        cbt_ref[...] = jnp.dot(i2, s_ref[...],
                               preferred_element_type=jnp.float32)


def _gemv(zf3, Wt, bl1c, Wl2, bl2r, S):
    zspec = [pl.BlockSpec((1, 1, BK),
                          (lambda j: (lambda k: (NSTREAM * k + j, 0, 0)))(j))
             for j in range(NSTREAM)]
    wspec = [pl.BlockSpec((INTER, BK),
                          (lambda j: (lambda k: (0, NSTREAM * k + j)))(j))
             for j in range(NSTREAM)]
    return pl.pallas_call(
        _gemv_body,
        grid=(NBK,),
        in_specs=zspec + wspec + [
            pl.BlockSpec((INTER, 1), lambda k: (0, 0)),
            pl.BlockSpec((INTER, N_FEAT), lambda k: (0, 0)),
            pl.BlockSpec((1, N_FEAT), lambda k: (0, 0)),
            pl.BlockSpec((N_FEAT, ROI * ROI), lambda k: (0, 0)),
        ],
        out_specs=(
            pl.BlockSpec((INTER, 1), lambda k: (0, 0)),
            pl.BlockSpec((1, ROI * ROI), lambda k: (0, 0)),
        ),
        out_shape=(
            jax.ShapeDtypeStruct((INTER, 1), jnp.float32),
            jax.ShapeDtypeStruct((1, ROI * ROI), jnp.float32),
        ),
    )(zf3, zf3, zf3, zf3, Wt, Wt, Wt, Wt, bl1c, Wl2, bl2r, S)


def kernel(x, edge_index, W1, b1, Wl1, bl1, Wl2, bl2):
    S = jnp.asarray(_S_np)
    src = edge_index[0]
    dst = edge_index[1]
    Cflat, deg2 = _sc_count(src, dst)
    C2 = Cflat.reshape(2 * N_NODES, N_NODES)
    dc = (deg2[:N_NODES] + deg2[N_NODES:]).reshape(N_NODES, 1)
    z, xs, zs = _dense(dc, x, W1, b1.reshape(1, -1), C2, S)
    zf3 = z.reshape(NBK * NSTREAM, 1, BK)
    Wt = Wl1.T
    i1, cbt = _gemv(zf3, Wt, bl1.reshape(-1, 1), Wl2,
                    bl2.reshape(1, -1), S)
    x_matrix = xs.reshape(N_NODES, ROI, ROI)
    z_matrix = zs.reshape(N_NODES, ROI, ROI)
    return (x_matrix, z_matrix, z_matrix, i1.reshape(INTER),
            cbt.reshape(ROI, ROI))


# speedup vs baseline: 1.0358x; 1.0183x over previous
import numpy as np
import jax
import jax.numpy as jnp
from jax import lax
from jax.experimental import pallas as pl
from jax.experimental.pallas import tpu as pltpu
from jax.experimental.pallas import tpu_sc as plsc

N_NODES = 1024
N_FEAT = 496
INTER = 64
N_EDGES = 65536
ROI = 32

_iu0, _iu1 = np.triu_indices(ROI, k=1)
_S_np = np.zeros((N_FEAT, ROI * ROI), np.float32)
_S_np[np.arange(N_FEAT), _iu0 * ROI + _iu1] = 1.0
_S_np[np.arange(N_FEAT), _iu1 * ROI + _iu0] = 1.0


ROWS_PER_W = N_NODES // 16
E_HALF = N_EDGES // 2
N_CHUNKS = 4
CH = E_HALF // N_CHUNKS
UNROLL = 4


def _sc_count_body(src_hbm, dst_hbm, out_hbm, deg_hbm, src_v, dst_v, acc_v,
                   deg_v):
    cid = lax.axis_index("c")
    sid = lax.axis_index("s")
    lo = sid * ROWS_PER_W
    ebase = cid * E_HALF
    zeros16 = jnp.zeros((16,), jnp.float32)
    ones16 = jnp.ones((16,), jnp.float32)

    def zero_row(i, carry):
        for c in range(N_NODES // 16):
            acc_v[pl.ds(i * N_NODES + c * 16, 16)] = zeros16
        return carry

    lax.fori_loop(0, ROWS_PER_W, zero_row, 0)
    for c in range(ROWS_PER_W // 16):
        deg_v[pl.ds(c * 16, 16)] = zeros16

    for ck in range(N_CHUNKS):
        pltpu.sync_copy(src_hbm.at[pl.ds(ebase + ck * CH, CH)], src_v)
        pltpu.sync_copy(dst_hbm.at[pl.ds(ebase + ck * CH, CH)], dst_v)

        def body(i, carry):
            for u in range(UNROLL):
                s = src_v[pl.ds(i * (16 * UNROLL) + u * 16, 16)]
                d = dst_v[pl.ds(i * (16 * UNROLL) + u * 16, 16)]
                rel = d - lo
                m = (rel >= 0) & (rel < ROWS_PER_W)
                rc = jnp.where(m, rel, 0)
                flat = rc * N_NODES + jnp.where(m, s, 0)
                plsc.addupdate_scatter(acc_v, [flat], ones16, mask=m)
                plsc.addupdate_scatter(deg_v, [rc], ones16, mask=m)
            return carry

        lax.fori_loop(0, CH // (16 * UNROLL), body, 0)

    pltpu.sync_copy(
        acc_v,
        out_hbm.at[pl.ds((cid * N_NODES + lo) * N_NODES,
                         ROWS_PER_W * N_NODES)])
    pltpu.sync_copy(
        deg_v, deg_hbm.at[pl.ds(cid * N_NODES + lo, ROWS_PER_W)])


_SC_COUNT_CACHE = []


def _sc_count(src, dst):
    if not _SC_COUNT_CACHE:
        _SC_COUNT_CACHE.append(pl.kernel(
            _sc_count_body,
            out_type=(
                jax.ShapeDtypeStruct((2 * N_NODES * N_NODES,), jnp.float32),
                jax.ShapeDtypeStruct((2 * N_NODES,), jnp.float32),
            ),
            mesh=plsc.VectorSubcoreMesh(core_axis_name="c", subcore_axis_name="s"),
            compiler_params=pltpu.CompilerParams(needs_layout_passes=False),
            scratch_types=[
                pltpu.VMEM((CH,), jnp.int32),
                pltpu.VMEM((CH,), jnp.int32),
                pltpu.VMEM((ROWS_PER_W * N_NODES,), jnp.float32),
                pltpu.VMEM((ROWS_PER_W,), jnp.float32),
            ],
        ))
    return _SC_COUNT_CACHE[0](src, dst)


def _pre_body(x_ref, w1_ref, s_ref, xw_ref, xs_ref):
    k = pl.program_id(0)
    x = x_ref[...]
    xw_ref[...] = jnp.dot(x, w1_ref[...], preferred_element_type=jnp.float32)
    xs_ref[...] = jnp.dot(x, s_ref[...], preferred_element_type=jnp.float32)


def _pre(x, W1, S):
    return pl.pallas_call(
        _pre_body,
        grid=(4,),
        in_specs=[
            pl.BlockSpec((N_NODES // 4, N_FEAT), lambda k: (k, 0)),
            pl.BlockSpec((N_FEAT, N_FEAT), lambda k: (0, 0)),
            pl.BlockSpec((N_FEAT, ROI * ROI), lambda k: (0, 0)),
        ],
        out_specs=(
            pl.BlockSpec((N_NODES // 4, N_FEAT), lambda k: (k, 0)),
            pl.BlockSpec((N_NODES // 4, ROI * ROI), lambda k: (k, 0)),
        ),
        out_shape=(
            jax.ShapeDtypeStruct((N_NODES, N_FEAT), jnp.float32),
            jax.ShapeDtypeStruct((N_NODES, ROI * ROI), jnp.float32),
        ),
    )(x, W1, S)


RB = 128
NRB = N_NODES // RB


def _dense_body(dcf_ref, xwf_ref, b1_ref, c0_ref, c1_ref,
                dc_ref, s_ref, z_ref, zs_ref, xws_ref):
    k = pl.program_id(0)

    @pl.when(k == 0)
    def _():
        dinv_all = lax.rsqrt(dcf_ref[...] + 1.0)
        xws_ref[...] = xwf_ref[...] * dinv_all

    C_blk = c0_ref[...] + c1_ref[...]
    dinv_col = lax.rsqrt(dc_ref[...] + 1.0)
    y = (jnp.dot(C_blk, xws_ref[...], preferred_element_type=jnp.float32)
         + xws_ref[pl.ds(k * RB, RB), :]) * dinv_col + b1_ref[...]
    z = jax.nn.sigmoid(y)
    z_ref[...] = z
    zs_ref[...] = jnp.dot(z, s_ref[...], preferred_element_type=jnp.float32)


def _dense(dc, xw, b1r, C2, S):
    return pl.pallas_call(
        _dense_body,
        grid=(NRB,),
        in_specs=[
            pl.BlockSpec((N_NODES, 1), lambda k: (0, 0)),
            pl.BlockSpec((N_NODES, N_FEAT), lambda k: (0, 0)),
            pl.BlockSpec((1, N_FEAT), lambda k: (0, 0)),
            pl.BlockSpec((RB, N_NODES), lambda k: (k, 0)),
            pl.BlockSpec((RB, N_NODES), lambda k: (k + NRB, 0)),
            pl.BlockSpec((RB, 1), lambda k: (k, 0)),
            pl.BlockSpec((N_FEAT, ROI * ROI), lambda k: (0, 0)),
        ],
        out_specs=(
            pl.BlockSpec((RB, N_FEAT), lambda k: (k, 0)),
            pl.BlockSpec((RB, ROI * ROI), lambda k: (k, 0)),
        ),
        out_shape=(
            jax.ShapeDtypeStruct((N_NODES, N_FEAT), jnp.float32),
            jax.ShapeDtypeStruct((N_NODES, ROI * ROI), jnp.float32),
        ),
        scratch_shapes=[pltpu.VMEM((N_NODES, N_FEAT), jnp.float32)],
    )(dc, xw, b1r, C2, C2, dc, S)


KF = N_NODES * N_FEAT
NBK = 16
NSTREAM = 4
BK = KF // (NBK * NSTREAM)


def _gemv_body(z0_ref, z1_ref, z2_ref, z3_ref,
               w0_ref, w1_ref, w2_ref, w3_ref,
               bl1_ref, wl2_ref, bl2_ref, s_ref,
               i1_ref, cbt_ref):
    k = pl.program_id(0)

    @pl.when(k == 0)
    def _():
        i1_ref[...] = jnp.zeros_like(i1_ref)

    acc = i1_ref[...]
    for zr, wr in ((z0_ref, w0_ref), (z1_ref, w1_ref),
                   (z2_ref, w2_ref), (z3_ref, w3_ref)):
        acc = acc + jax.lax.dot_general(
            wr[...], zr[0],
            dimension_numbers=(((1,), (1,)), ((), ())),
            preferred_element_type=jnp.float32)
    i1_ref[...] = acc

    @pl.when(k == NBK - 1)
    def _():
        i1 = jax.nn.sigmoid(i1_ref[...] + bl1_ref[...])
        i1_ref[...] = i1
        i2 = jax.nn.sigmoid(
            jax.lax.dot_general(
                i1, wl2_ref[...],
                dimension_numbers=(((0,), (0,)), ((), ())),
                preferred_element_type=jnp.float32)
            + bl2_ref[...])
        cbt_ref[...] = jnp.dot(i2, s_ref[...],
                               preferred_element_type=jnp.float32)


def _gemv(zf3, Wt, bl1c, Wl2, bl2r, S):
    zspec = [pl.BlockSpec((1, 1, BK),
                          (lambda j: (lambda k: (NSTREAM * k + j, 0, 0)))(j))
             for j in range(NSTREAM)]
    wspec = [pl.BlockSpec((INTER, BK),
                          (lambda j: (lambda k: (0, NSTREAM * k + j)))(j))
             for j in range(NSTREAM)]
    return pl.pallas_call(
        _gemv_body,
        grid=(NBK,),
        in_specs=zspec + wspec + [
            pl.BlockSpec((INTER, 1), lambda k: (0, 0)),
            pl.BlockSpec((INTER, N_FEAT), lambda k: (0, 0)),
            pl.BlockSpec((1, N_FEAT), lambda k: (0, 0)),
            pl.BlockSpec((N_FEAT, ROI * ROI), lambda k: (0, 0)),
        ],
        out_specs=(
            pl.BlockSpec((INTER, 1), lambda k: (0, 0)),
            pl.BlockSpec((1, ROI * ROI), lambda k: (0, 0)),
        ),
        out_shape=(
            jax.ShapeDtypeStruct((INTER, 1), jnp.float32),
            jax.ShapeDtypeStruct((1, ROI * ROI), jnp.float32),
        ),
    )(zf3, zf3, zf3, zf3, Wt, Wt, Wt, Wt, bl1c, Wl2, bl2r, S)


def kernel(x, edge_index, W1, b1, Wl1, bl1, Wl2, bl2):
    S = jnp.asarray(_S_np)
    src = edge_index[0]
    dst = edge_index[1]
    Cflat, deg2 = _sc_count(src, dst)
    xw, xs = _pre(x, W1, S)
    C2 = Cflat.reshape(2 * N_NODES, N_NODES)
    dc = (deg2[:N_NODES] + deg2[N_NODES:]).reshape(N_NODES, 1)
    z, zs = _dense(dc, xw, b1.reshape(1, -1), C2, S)
    zf3 = z.reshape(NBK * NSTREAM, 1, BK)
    Wt = Wl1.T
    i1, cbt = _gemv(zf3, Wt, bl1.reshape(-1, 1), Wl2,
                    bl2.reshape(1, -1), S)
    x_matrix = xs.reshape(N_NODES, ROI, ROI)
    z_matrix = zs.reshape(N_NODES, ROI, ROI)
    return (x_matrix, z_matrix, z_matrix, i1.reshape(INTER),
            cbt.reshape(ROI, ROI))
